# baseline (speedup 1.0000x reference)
import functools

import jax
import jax.numpy as jnp
from jax import lax
from jax.experimental import pallas as pl
from jax.experimental.pallas import tpu as pltpu
from jax.experimental.pallas import tpu_sc as plsc

N = 10000
E = 320000
D = 128
H = 64

NPAD = 10240
TW = 80
GW = 128
NTILES = 32
EPT = 10240
EPAD = EPT * NTILES
RPT = NPAD // 16

SCH = 128
SNC = EPT // SCH
SIG = 16
SNG = SNC // SIG

ECH = 128
EHALF = EPAD // 2
EQ = EPAD // 4
ERPT = EQ // NTILES
EWS = ERPT // ECH
EIG = 10
ENG = EWS // EIG

f32 = jnp.float32

_SC_PARAMS = pltpu.CompilerParams(use_tc_tiling_on_sc=False)


def _mktab(y, w):
    m = y.shape[0]
    return jnp.concatenate(
        [y, jnp.ones((m, 1), f32), jnp.zeros((m, w - H - 1), f32)], axis=1)


def _node1_body(x_ref, wl_ref, wr_ref, bl_ref, yt_ref, r_ref):
    x = x_ref[...]
    y = jnp.dot(x, wl_ref[...], preferred_element_type=f32)
    yt_ref[...] = _mktab(y, TW)
    r_ref[...] = jnp.dot(x, wr_ref[...], preferred_element_type=f32) + bl_ref[...]


def _node1(xp, Wl1, Wr1, bl1):
    G = NPAD // 512
    return pl.pallas_call(
        _node1_body,
        grid=(G,),
        in_specs=[
            pl.BlockSpec((512, D), lambda i: (i, 0)),
            pl.BlockSpec((D, H), lambda i: (0, 0)),
            pl.BlockSpec((D, H), lambda i: (0, 0)),
            pl.BlockSpec((1, H), lambda i: (0, 0)),
        ],
        out_specs=[
            pl.BlockSpec((512, TW), lambda i: (i, 0)),
            pl.BlockSpec((512, H), lambda i: (i, 0)),
        ],
        out_shape=[
            jax.ShapeDtypeStruct((NPAD, TW), f32),
            jax.ShapeDtypeStruct((NPAD, H), f32),
        ],
    )(xp, Wl1, Wr1, bl1)


def _comb_body(a0_ref, a1_ref, r_ref, wl_ref, wr_ref, bl_ref, dep_ref,
               yt_ref, r2_ref):
    a = a0_ref[...] + a1_ref[...]
    deg = jnp.maximum(a[:, H:H + 1], 1.0)
    h = jnp.maximum(a[:, :H] / deg + r_ref[...], 0.0)
    y2 = jnp.dot(h, wl_ref[...], preferred_element_type=f32)
    yt_ref[...] = _mktab(y2, TW)
    r2_ref[...] = jnp.dot(h, wr_ref[...], preferred_element_type=f32) + bl_ref[...]


def _comb1(acc, r1, Wl2, Wr2, bl2, dep):
    G = NPAD // 512
    return pl.pallas_call(
        _comb_body,
        grid=(G,),
        in_specs=[
            pl.BlockSpec((512, TW), lambda i: (i, 0)),
            pl.BlockSpec((512, TW), lambda i: (i + NPAD // 512, 0)),
            pl.BlockSpec((512, H), lambda i: (i, 0)),
            pl.BlockSpec((H, H), lambda i: (0, 0)),
            pl.BlockSpec((H, H), lambda i: (0, 0)),
            pl.BlockSpec((1, H), lambda i: (0, 0)),
            pl.BlockSpec((16, 2 * H), lambda i: (0, 0)),
        ],
        out_specs=[
            pl.BlockSpec((512, TW), lambda i: (i, 0)),
            pl.BlockSpec((512, H), lambda i: (i, 0)),
        ],
        out_shape=[
            jax.ShapeDtypeStruct((NPAD, TW), f32),
            jax.ShapeDtypeStruct((NPAD, H), f32),
        ],
    )(acc, acc, r1, Wl2, Wr2, bl2, dep)


def _comb2_body(a0_ref, a1_ref, r_ref, wo_ref, dep_ref, g_ref):
    a = a0_ref[...] + a1_ref[...]
    deg = jnp.maximum(a[:, H:H + 1], 1.0)
    h = jnp.maximum(a[:, :H] / deg + r_ref[...], 0.0)
    g_ref[...] = jnp.dot(h, wo_ref[...], preferred_element_type=f32)


def _comb2(acc, r2, Wo1, dep):
    G = NPAD // 512
    return pl.pallas_call(
        _comb2_body,
        grid=(G,),
        in_specs=[
            pl.BlockSpec((512, TW), lambda i: (i, 0)),
            pl.BlockSpec((512, TW), lambda i: (i + NPAD // 512, 0)),
            pl.BlockSpec((512, H), lambda i: (i, 0)),
            pl.BlockSpec((H, H), lambda i: (0, 0)),
            pl.BlockSpec((16, 2 * H), lambda i: (0, 0)),
        ],
        out_specs=pl.BlockSpec((512, H), lambda i: (i, 0)),
        out_shape=jax.ShapeDtypeStruct((NPAD, H), f32),
    )(acc, acc, r2, Wo1, dep)


def _mlp_body(ea_ref, eb_ref, we1_ref, be1_ref, we2_ref, wo1_ref,
              bo1_ref, be2_ref, z_ref):
    w2o = jnp.dot(we2_ref[...], wo1_ref[...], preferred_element_type=f32)
    cvec = jnp.dot(be2_ref[...], wo1_ref[...], preferred_element_type=f32) + bo1_ref[...]

    def half(ref):
        t = jnp.maximum(
            lax.dot_general(ref[...], we1_ref[...],
                            (((0,), (0,)), ((), ())),
                            preferred_element_type=f32)
            + be1_ref[...], 0.0)
        return jnp.dot(t, w2o, preferred_element_type=f32) + cvec

    z_ref[...] = jnp.concatenate(
        [half(ea_ref), half(eb_ref)], axis=1).astype(jnp.bfloat16)


def _edge_mlp(eaT, We1, be1, We2, Wo1, bo1, be2, k):
    BE = 2048
    G = EQ // BE
    c1 = k * G
    c2 = 2 * G + k * G
    return pl.pallas_call(
        _mlp_body,
        grid=(G,),
        in_specs=[
            pl.BlockSpec((16, BE), lambda i, c=c1: (0, i + c)),
            pl.BlockSpec((16, BE), lambda i, c=c2: (0, i + c)),
            pl.BlockSpec((16, H), lambda i: (0, 0)),
            pl.BlockSpec((1, H), lambda i: (0, 0)),
            pl.BlockSpec((H, H), lambda i: (0, 0)),
            pl.BlockSpec((H, H), lambda i: (0, 0)),
            pl.BlockSpec((1, H), lambda i: (0, 0)),
            pl.BlockSpec((1, H), lambda i: (0, 0)),
        ],
        out_specs=pl.BlockSpec((BE, 2 * H), lambda i: (i, 0)),
        out_shape=jax.ShapeDtypeStruct((EQ, 2 * H), jnp.bfloat16),
    )(eaT, eaT, We1, be1, We2, Wo1, bo1, be2)


def _out_body(z_ref, efp_ref, w2_ref, bo2_ref, o_ref):
    z = jnp.maximum(z_ref[...].astype(f32) + efp_ref[...], 0.0)
    o2 = jnp.dot(z, w2_ref[...], preferred_element_type=f32) + bo2_ref[0, 0]
    o2 = jax.nn.sigmoid(o2)
    m = z.shape[0]
    i = pl.program_id(0)
    off = pl.multiple_of(i * m, 1024)
    o_ref[pl.ds(off, m)] = jnp.reshape(o2[:, 0:1], (m,))
    o_ref[pl.ds(EQ + off, m)] = jnp.reshape(o2[:, 1:2], (m,))


def _out_stage(zp2, ef2, W2stack, bo2):
    BE = 2048
    G = EQ // BE
    return pl.pallas_call(
        _out_body,
        grid=(G,),
        in_specs=[
            pl.BlockSpec((BE, 2 * H), lambda i: (i, 0)),
            pl.BlockSpec((BE, 2 * H), lambda i: (i, 0)),
            pl.BlockSpec((2 * H, 2), lambda i: (0, 0)),
            pl.BlockSpec((1, 1), lambda i: (0, 0)),
        ],
        out_specs=pl.BlockSpec((2 * EQ,), lambda i: (0,)),
        out_shape=jax.ShapeDtypeStruct((2 * EQ,), f32),
    )(zp2, ef2, W2stack, bo2)


_MESH = dict(core_axis_name="c", subcore_axis_name="s")


def _seg_sum(table, srcm, dstm, ztab):
    mesh = plsc.VectorSubcoreMesh(**_MESH)

    @functools.partial(
        pl.kernel,
        out_type=jax.ShapeDtypeStruct((2 * NPAD, TW), f32),
        mesh=mesh,
        compiler_params=_SC_PARAMS,
        scratch_types=[
            pltpu.VMEM_SHARED((NPAD, TW), f32),
            pltpu.VMEM((SIG, SCH), jnp.int32),
            pltpu.VMEM((SIG, SCH), jnp.int32),
            pltpu.VMEM((SCH, TW), f32),
            pltpu.VMEM((SCH, TW), f32),
            pltpu.SemaphoreType.DMA,
            pltpu.SemaphoreType.DMA,
        ],
    )
    def k(table_hbm, src_hbm, dst_hbm, z_hbm, out_hbm,
          acc_sh, sidx, didx, rows0, rows1, sem0, sem1):
        c = lax.axis_index("c")
        s = lax.axis_index("s")
        r0 = s * RPT
        pltpu.sync_copy(z_hbm.at[pl.ds(r0, RPT)], acc_sh.at[pl.ds(r0, RPT)])
        wid = c * 16 + s
        cb = wid * SNC
        plsc.subcore_barrier()

        rbufs = (rows0, rows1)
        sems = (sem0, sem1)

        def grp(gi, carry):
            pltpu.sync_copy(src_hbm.at[pl.ds(cb + gi * SIG, SIG)], sidx)
            pltpu.sync_copy(dst_hbm.at[pl.ds(cb + gi * SIG, SIG)], didx)
            copies = [None, None]
            copies[0] = pltpu.async_copy(
                table_hbm.at[sidx.at[0]], rbufs[0], sems[0])
            for j in range(SIG):
                p = j % 2
                copies[p].wait()
                if j + 1 < SIG:
                    q = (j + 1) % 2
                    copies[q] = pltpu.async_copy(
                        table_hbm.at[sidx.at[j + 1]], rbufs[q], sems[q])
                pltpu.sync_copy(rbufs[p], acc_sh.at[didx.at[j]], add=True)
            return carry

        lax.fori_loop(0, SNG, grp, 0)
        plsc.subcore_barrier()
        pltpu.sync_copy(acc_sh.at[pl.ds(r0, RPT)],
                        out_hbm.at[pl.ds(c * NPAD + r0, RPT)])

    return k(table, srcm, dstm, ztab)


def _edge_feats(g, srcm, dstm):
    mesh = plsc.VectorSubcoreMesh(**_MESH)

    @functools.partial(
        pl.kernel,
        out_type=jax.ShapeDtypeStruct((EQ, 2 * H), f32),
        mesh=mesh,
        compiler_params=_SC_PARAMS,
        scratch_types=[
            pltpu.VMEM((2 * EWS, ECH), jnp.int32),
            pltpu.VMEM((2 * EWS, ECH), jnp.int32),
            pltpu.VMEM((ECH, H), f32),
            pltpu.VMEM((ECH, H), f32),
            pltpu.VMEM((ECH, H), f32),
            pltpu.VMEM((ECH, H), f32),
            pltpu.VMEM((ECH, H), f32),
            pltpu.SemaphoreType.DMA,
            pltpu.SemaphoreType.DMA,
            pltpu.SemaphoreType.DMA,
            pltpu.SemaphoreType.DMA,
        ],
    )
    def kk(g_hbm, src_hbm, dst_hbm, out_hbm,
           sidx, didx, ra0, rb0, ra1, rb1, rc, sa0, sb0, sa1, sb1):
        c = lax.axis_index("c")
        s = lax.axis_index("s")
        wid = c * 16 + s
        r0 = wid * ERPT
        pltpu.sync_copy(src_hbm.at[pl.ds(wid * 2 * EWS, 2 * EWS)], sidx)
        pltpu.sync_copy(dst_hbm.at[pl.ds(wid * 2 * EWS, 2 * EWS)], didx)

        ras = (ra0, ra1)
        rbs = (rb0, rb1)
        sas = (sa0, sa1)
        sbs = (sb0, sb1)

        def gath(row, p):
            return (pltpu.async_copy(g_hbm.at[sidx.at[row]], ras[p], sas[p]),
                    pltpu.async_copy(g_hbm.at[didx.at[row]], rbs[p], sbs[p]))

        def consume(w, p, co):
            ra = ras[p]
            rb = rbs[p]

            def row(i, carry2):
                for qq in range(H // 16):
                    av = ra[i, pl.ds(qq * 16, 16)]
                    bv = rb[i, pl.ds(qq * 16, 16)]
                    rc[i, pl.ds(qq * 16, 16)] = av + bv
                return carry2

            lax.fori_loop(0, ECH, row, 0)
            pltpu.sync_copy(
                rc, out_hbm.at[pl.ds(r0 + w * ECH, ECH), pl.ds(co, H)])

        for si in range(2):
            so = si * EWS
            co = si * H
            cp0 = gath(so, 0)
            cp1 = gath(so + 1, 1)

            def pair(p, carry, so=so, co=co):
                w0 = 2 * p
                for x in cp0:
                    x.wait()
                consume(w0, 0, co)
                nxt0 = jnp.minimum(so + w0 + 2, so + EWS - 1)
                c0 = gath(nxt0, 0)
                for x in cp1:
                    x.wait()
                consume(w0 + 1, 1, co)
                nxt1 = jnp.minimum(so + w0 + 3, so + EWS - 1)
                c1 = gath(nxt1, 1)
                return carry

            lax.fori_loop(0, EWS // 2, pair, 0)
            for p in range(2):
                pltpu.make_async_copy(
                    g_hbm.at[sidx.at[so]], ras[p], sas[p]).wait()
                pltpu.make_async_copy(
                    g_hbm.at[didx.at[so]], rbs[p], sbs[p]).wait()

    return kk(g, srcm, dstm)


def kernel(x, edge_index, edge_attr, We1, be1, We2, be2,
           Wl1, bl1, Wr1, Wl2, bl2, Wr2, Wo1, bo1, Wo2, bo2):
    src = edge_index[0].astype(jnp.int32)
    dst = edge_index[1].astype(jnp.int32)
    npadv = EPAD - E
    padv = N + (jnp.arange(npadv, dtype=jnp.int32) % 16)
    srcp = jnp.concatenate([src, padv])
    dstp = jnp.concatenate([dst, padv])
    nrh = EPAD // ECH // 2
    rows = []
    for k in range(2):
        for wid in range(NTILES):
            base = k * (EQ // ECH) + wid * EWS
            rows.extend(range(base, base + EWS))
            rows.extend(range(nrh + base, nrh + base + EWS))
    perm = jnp.asarray(rows, dtype=jnp.int32)
    srcm_p = jnp.take(srcp.reshape(EPAD // ECH, ECH), perm, axis=0)
    dstm_p = jnp.take(dstp.reshape(EPAD // ECH, ECH), perm, axis=0)
    srcm_s = srcm_p
    dstm_s = dstm_p
    srcm_e0 = srcm_p[:nrh]
    dstm_e0 = dstm_p[:nrh]
    srcm_e1 = srcm_p[nrh:]
    dstm_e1 = dstm_p[nrh:]

    xp = jnp.pad(x, ((0, NPAD - N), (0, 0)))
    ztab = jnp.zeros((NPAD, TW), f32)

    bl1r = bl1.reshape(1, H)
    bl2r = bl2.reshape(1, H)
    be1r = be1.reshape(1, H)
    be2r = be2.reshape(1, H)
    bo1r = bo1.reshape(1, H)
    bo2r = bo2.reshape(1, 1)

    eaT = jnp.pad(edge_attr.T, ((0, 0), (0, EPAD - E)))
    zp0 = _edge_mlp(eaT, We1, be1r, We2, Wo1, bo1r, be2r, 0)
    zp1 = _edge_mlp(eaT, We1, be1r, We2, Wo1, bo1r, be2r, 1)
    W2stack = jnp.zeros((2 * H, 2), f32)
    W2stack = W2stack.at[:H, 0].set(Wo2[:, 0]).at[H:, 1].set(Wo2[:, 0])

    y1t, r1 = _node1(xp, Wl1, Wr1, bl1r)
    acc1 = _seg_sum(y1t, srcm_s, dstm_s, ztab)
    y2t, r2 = _comb1(acc1, r1, Wl2, Wr2, bl2r, zp0)
    acc2 = _seg_sum(y2t, srcm_s, dstm_s, ztab)
    g = _comb2(acc2, r2, Wo1, zp1)
    ef0 = _edge_feats(g, srcm_e0, dstm_e0)
    ef1 = _edge_feats(g, srcm_e1, dstm_e1)
    o0 = _out_stage(zp0, ef0, W2stack, bo2r)
    o1 = _out_stage(zp1, ef1, W2stack, bo2r)
    return jnp.concatenate(
        [o0[:EQ], o1[:EQ], o0[EQ:], o1[EQ:EQ + (E - EHALF - EQ)]])


# speedup vs baseline: 1.0136x; 1.0136x over previous
import functools

import jax
import jax.numpy as jnp
from jax import lax
from jax.experimental import pallas as pl
from jax.experimental.pallas import tpu as pltpu
from jax.experimental.pallas import tpu_sc as plsc

N = 10000
E = 320000
D = 128
H = 64

NPAD = 10240
TW = 80
GW = 128
NTILES = 32
EPT = 10240
EPAD = EPT * NTILES
RPT = NPAD // 16

SCH = 128
SNC = EPT // SCH
SIG = 16
SNG = SNC // SIG

ECH = 128
EHALF = EPAD // 2
EQ = EPAD // 4
ERPT = EQ // NTILES
EWS = ERPT // ECH
EIG = 10
ENG = EWS // EIG

f32 = jnp.float32

_SC_PARAMS = pltpu.CompilerParams(use_tc_tiling_on_sc=False)


def _mktab(y, w):
    m = y.shape[0]
    return jnp.concatenate(
        [y, jnp.ones((m, 1), f32), jnp.zeros((m, w - H - 1), f32)], axis=1)


def _node1_body(x_ref, wl_ref, wr_ref, bl_ref, yt_ref, r_ref):
    x = x_ref[...]
    y = jnp.dot(x, wl_ref[...], preferred_element_type=f32)
    yt_ref[...] = _mktab(y, TW)
    r_ref[...] = jnp.dot(x, wr_ref[...], preferred_element_type=f32) + bl_ref[...]


def _node1(xp, Wl1, Wr1, bl1):
    G = NPAD // 512
    return pl.pallas_call(
        _node1_body,
        grid=(G,),
        in_specs=[
            pl.BlockSpec((512, D), lambda i: (i, 0)),
            pl.BlockSpec((D, H), lambda i: (0, 0)),
            pl.BlockSpec((D, H), lambda i: (0, 0)),
            pl.BlockSpec((1, H), lambda i: (0, 0)),
        ],
        out_specs=[
            pl.BlockSpec((512, TW), lambda i: (i, 0)),
            pl.BlockSpec((512, H), lambda i: (i, 0)),
        ],
        out_shape=[
            jax.ShapeDtypeStruct((NPAD, TW), f32),
            jax.ShapeDtypeStruct((NPAD, H), f32),
        ],
    )(xp, Wl1, Wr1, bl1)


def _comb_body(a0_ref, a1_ref, r_ref, wl_ref, wr_ref, bl_ref, dep_ref,
               yt_ref, r2_ref):
    a = a0_ref[...] + a1_ref[...]
    deg = jnp.maximum(a[:, H:H + 1], 1.0)
    h = jnp.maximum(a[:, :H] / deg + r_ref[...], 0.0)
    y2 = jnp.dot(h, wl_ref[...], preferred_element_type=f32)
    yt_ref[...] = _mktab(y2, TW)
    r2_ref[...] = jnp.dot(h, wr_ref[...], preferred_element_type=f32) + bl_ref[...]


def _comb1(acc, r1, Wl2, Wr2, bl2, dep):
    G = NPAD // 512
    return pl.pallas_call(
        _comb_body,
        grid=(G,),
        in_specs=[
            pl.BlockSpec((512, TW), lambda i: (i, 0)),
            pl.BlockSpec((512, TW), lambda i: (i + NPAD // 512, 0)),
            pl.BlockSpec((512, H), lambda i: (i, 0)),
            pl.BlockSpec((H, H), lambda i: (0, 0)),
            pl.BlockSpec((H, H), lambda i: (0, 0)),
            pl.BlockSpec((1, H), lambda i: (0, 0)),
            pl.BlockSpec((16, 2 * H), lambda i: (0, 0)),
        ],
        out_specs=[
            pl.BlockSpec((512, TW), lambda i: (i, 0)),
            pl.BlockSpec((512, H), lambda i: (i, 0)),
        ],
        out_shape=[
            jax.ShapeDtypeStruct((NPAD, TW), f32),
            jax.ShapeDtypeStruct((NPAD, H), f32),
        ],
    )(acc, acc, r1, Wl2, Wr2, bl2, dep)


def _comb2_body(a0_ref, a1_ref, r_ref, wo_ref, dep_ref, g_ref):
    a = a0_ref[...] + a1_ref[...]
    deg = jnp.maximum(a[:, H:H + 1], 1.0)
    h = jnp.maximum(a[:, :H] / deg + r_ref[...], 0.0)
    g_ref[...] = jnp.dot(h, wo_ref[...], preferred_element_type=f32)


def _comb2(acc, r2, Wo1, dep):
    G = NPAD // 512
    return pl.pallas_call(
        _comb2_body,
        grid=(G,),
        in_specs=[
            pl.BlockSpec((512, TW), lambda i: (i, 0)),
            pl.BlockSpec((512, TW), lambda i: (i + NPAD // 512, 0)),
            pl.BlockSpec((512, H), lambda i: (i, 0)),
            pl.BlockSpec((H, H), lambda i: (0, 0)),
            pl.BlockSpec((16, 2 * H), lambda i: (0, 0)),
        ],
        out_specs=pl.BlockSpec((512, H), lambda i: (i, 0)),
        out_shape=jax.ShapeDtypeStruct((NPAD, H), f32),
    )(acc, acc, r2, Wo1, dep)


def _mlp_body(ea_ref, eb_ref, we1_ref, be1_ref, we2_ref, wo1_ref,
              bo1_ref, be2_ref, z_ref):
    w2o = jnp.dot(we2_ref[...], wo1_ref[...], preferred_element_type=f32)
    cvec = jnp.dot(be2_ref[...], wo1_ref[...], preferred_element_type=f32) + bo1_ref[...]

    def half(ref):
        t = jnp.maximum(
            lax.dot_general(ref[...], we1_ref[...],
                            (((0,), (0,)), ((), ())),
                            preferred_element_type=f32)
            + be1_ref[...], 0.0)
        return jnp.dot(t, w2o, preferred_element_type=f32) + cvec

    z_ref[...] = jnp.concatenate(
        [half(ea_ref), half(eb_ref)], axis=1).astype(jnp.bfloat16)


def _edge_mlp(eaT, We1, be1, We2, Wo1, bo1, be2, k):
    BE = 2048
    G = EQ // BE
    c1 = k * G
    c2 = 2 * G + k * G
    return pl.pallas_call(
        _mlp_body,
        grid=(G,),
        in_specs=[
            pl.BlockSpec((16, BE), lambda i, c=c1: (0, i + c)),
            pl.BlockSpec((16, BE), lambda i, c=c2: (0, i + c)),
            pl.BlockSpec((16, H), lambda i: (0, 0)),
            pl.BlockSpec((1, H), lambda i: (0, 0)),
            pl.BlockSpec((H, H), lambda i: (0, 0)),
            pl.BlockSpec((H, H), lambda i: (0, 0)),
            pl.BlockSpec((1, H), lambda i: (0, 0)),
            pl.BlockSpec((1, H), lambda i: (0, 0)),
        ],
        out_specs=pl.BlockSpec((BE, 2 * H), lambda i: (i, 0)),
        out_shape=jax.ShapeDtypeStruct((EQ, 2 * H), jnp.bfloat16),
    )(eaT, eaT, We1, be1, We2, Wo1, bo1, be2)


def _out_body(z_ref, efp_ref, w2_ref, bo2_ref, o_ref):
    z = jnp.maximum(z_ref[...].astype(f32) + efp_ref[...], 0.0)
    o2 = jnp.dot(z, w2_ref[...], preferred_element_type=f32) + bo2_ref[0, 0]
    o2 = jax.nn.sigmoid(o2)
    m = z.shape[0]
    i = pl.program_id(0)
    off = pl.multiple_of(i * m, 1024)
    o_ref[pl.ds(off, m)] = jnp.reshape(o2[:, 0:1], (m,))
    o_ref[pl.ds(EQ + off, m)] = jnp.reshape(o2[:, 1:2], (m,))


def _out_stage(zp2, ef2, W2stack, bo2):
    BE = 2048
    G = EQ // BE
    return pl.pallas_call(
        _out_body,
        grid=(G,),
        in_specs=[
            pl.BlockSpec((BE, 2 * H), lambda i: (i, 0)),
            pl.BlockSpec((BE, 2 * H), lambda i: (i, 0)),
            pl.BlockSpec((2 * H, 2), lambda i: (0, 0)),
            pl.BlockSpec((1, 1), lambda i: (0, 0)),
        ],
        out_specs=pl.BlockSpec((2 * EQ,), lambda i: (0,)),
        out_shape=jax.ShapeDtypeStruct((2 * EQ,), f32),
    )(zp2, ef2, W2stack, bo2)


_MESH = dict(core_axis_name="c", subcore_axis_name="s")


def _seg_sum(table, srcm, dstm, ztab):
    mesh = plsc.VectorSubcoreMesh(**_MESH)

    @functools.partial(
        pl.kernel,
        out_type=jax.ShapeDtypeStruct((2 * NPAD, TW), f32),
        mesh=mesh,
        compiler_params=_SC_PARAMS,
        scratch_types=[
            pltpu.VMEM_SHARED((NPAD, TW), f32),
            pltpu.VMEM((SIG, SCH), jnp.int32),
            pltpu.VMEM((SIG, SCH), jnp.int32),
            pltpu.VMEM((SCH, TW), f32),
            pltpu.VMEM((SCH, TW), f32),
            pltpu.SemaphoreType.DMA,
            pltpu.SemaphoreType.DMA,
        ],
    )
    def k(table_hbm, src_hbm, dst_hbm, z_hbm, out_hbm,
          acc_sh, sidx, didx, rows0, rows1, sem0, sem1):
        c = lax.axis_index("c")
        s = lax.axis_index("s")
        r0 = s * RPT
        pltpu.sync_copy(z_hbm.at[pl.ds(r0, RPT)], acc_sh.at[pl.ds(r0, RPT)])
        wid = c * 16 + s
        cb = wid * SNC
        plsc.subcore_barrier()

        rbufs = (rows0, rows1)
        sems = (sem0, sem1)

        def grp(gi, carry):
            pltpu.sync_copy(src_hbm.at[pl.ds(cb + gi * SIG, SIG)], sidx)
            pltpu.sync_copy(dst_hbm.at[pl.ds(cb + gi * SIG, SIG)], didx)
            copies = [None, None]
            copies[0] = pltpu.async_copy(
                table_hbm.at[sidx.at[0]], rbufs[0], sems[0])
            for j in range(SIG):
                p = j % 2
                copies[p].wait()
                if j + 1 < SIG:
                    q = (j + 1) % 2
                    copies[q] = pltpu.async_copy(
                        table_hbm.at[sidx.at[j + 1]], rbufs[q], sems[q])
                pltpu.sync_copy(rbufs[p], acc_sh.at[didx.at[j]], add=True)
            return carry

        lax.fori_loop(0, SNG, grp, 0)
        plsc.subcore_barrier()
        pltpu.sync_copy(acc_sh.at[pl.ds(r0, RPT)],
                        out_hbm.at[pl.ds(c * NPAD + r0, RPT)])

    return k(table, srcm, dstm, ztab)


def _edge_feats(g, srcm, dstm):
    mesh = plsc.VectorSubcoreMesh(**_MESH)

    @functools.partial(
        pl.kernel,
        out_type=jax.ShapeDtypeStruct((EQ, 2 * H), f32),
        mesh=mesh,
        compiler_params=_SC_PARAMS,
        scratch_types=[
            pltpu.VMEM((2 * EWS, ECH), jnp.int32),
            pltpu.VMEM((2 * EWS, ECH), jnp.int32),
            pltpu.VMEM((ECH, H), f32),
            pltpu.VMEM((ECH, H), f32),
            pltpu.VMEM((ECH, H), f32),
            pltpu.VMEM((ECH, H), f32),
            pltpu.VMEM((ECH, H), f32),
            pltpu.SemaphoreType.DMA,
            pltpu.SemaphoreType.DMA,
            pltpu.SemaphoreType.DMA,
            pltpu.SemaphoreType.DMA,
        ],
    )
    def kk(g_hbm, src_hbm, dst_hbm, out_hbm,
           sidx, didx, ra0, rb0, ra1, rb1, rc, sa0, sb0, sa1, sb1):
        c = lax.axis_index("c")
        s = lax.axis_index("s")
        wid = c * 16 + s
        r0 = wid * ERPT
        pltpu.sync_copy(src_hbm.at[pl.ds(wid * 2 * EWS, 2 * EWS)], sidx)
        pltpu.sync_copy(dst_hbm.at[pl.ds(wid * 2 * EWS, 2 * EWS)], didx)

        ras = (ra0, ra1)
        rbs = (rb0, rb1)
        sas = (sa0, sa1)
        sbs = (sb0, sb1)

        def gath(row, p):
            return (pltpu.async_copy(g_hbm.at[sidx.at[row]], ras[p], sas[p]),
                    pltpu.async_copy(g_hbm.at[didx.at[row]], rbs[p], sbs[p]))

        def consume(w, p, co):
            ra = ras[p]
            rb = rbs[p]

            def row(i, carry2):
                for qq in range(H // 16):
                    av = ra[i, pl.ds(qq * 16, 16)]
                    bv = rb[i, pl.ds(qq * 16, 16)]
                    rc[i, pl.ds(qq * 16, 16)] = av + bv
                return carry2

            lax.fori_loop(0, ECH, row, 0)
            pltpu.sync_copy(
                rc, out_hbm.at[pl.ds(r0 + w * ECH, ECH), pl.ds(co, H)])

        for si in range(2):
            so = si * EWS
            co = si * H
            cp0 = gath(so, 0)
            cp1 = gath(so + 1, 1)

            def pair(p, carry, so=so, co=co):
                w0 = 2 * p
                for x in cp0:
                    x.wait()
                consume(w0, 0, co)
                nxt0 = jnp.minimum(so + w0 + 2, so + EWS - 1)
                c0 = gath(nxt0, 0)
                for x in cp1:
                    x.wait()
                consume(w0 + 1, 1, co)
                nxt1 = jnp.minimum(so + w0 + 3, so + EWS - 1)
                c1 = gath(nxt1, 1)
                return carry

            lax.fori_loop(0, EWS // 2, pair, 0)
            for p in range(2):
                pltpu.make_async_copy(
                    g_hbm.at[sidx.at[so]], ras[p], sas[p]).wait()
                pltpu.make_async_copy(
                    g_hbm.at[didx.at[so]], rbs[p], sbs[p]).wait()

    return kk(g, srcm, dstm)


def kernel(x, edge_index, edge_attr, We1, be1, We2, be2,
           Wl1, bl1, Wr1, Wl2, bl2, Wr2, Wo1, bo1, Wo2, bo2):
    src = edge_index[0].astype(jnp.int32)
    dst = edge_index[1].astype(jnp.int32)
    npadv = EPAD - E
    padv = N + (jnp.arange(npadv, dtype=jnp.int32) % 16)
    srcp = jnp.concatenate([src, padv])
    dstp = jnp.concatenate([dst, padv])
    srcm_s = srcp.reshape(EPAD // SCH, SCH)
    dstm_s = dstp.reshape(EPAD // SCH, SCH)
    srcm_e = srcp.reshape(EPAD // ECH, ECH)
    dstm_e = dstp.reshape(EPAD // ECH, ECH)
    nrh = EPAD // ECH // 2
    perms = []
    for k in range(2):
        rows = []
        for wid in range(NTILES):
            base = k * (EQ // ECH) + wid * EWS
            rows.extend(range(base, base + EWS))
            rows.extend(range(nrh + base, nrh + base + EWS))
        perms.append(jnp.asarray(rows, dtype=jnp.int32))
    srcm_e0 = jnp.take(srcm_e, perms[0], axis=0)
    dstm_e0 = jnp.take(dstm_e, perms[0], axis=0)
    srcm_e1 = jnp.take(srcm_e, perms[1], axis=0)
    dstm_e1 = jnp.take(dstm_e, perms[1], axis=0)

    xp = jnp.pad(x, ((0, NPAD - N), (0, 0)))
    ztab = jnp.zeros((NPAD, TW), f32)

    bl1r = bl1.reshape(1, H)
    bl2r = bl2.reshape(1, H)
    be1r = be1.reshape(1, H)
    be2r = be2.reshape(1, H)
    bo1r = bo1.reshape(1, H)
    bo2r = bo2.reshape(1, 1)

    eaT = jnp.pad(edge_attr.T, ((0, 0), (0, EPAD - E)))
    zp0 = _edge_mlp(eaT, We1, be1r, We2, Wo1, bo1r, be2r, 0)
    zp1 = _edge_mlp(eaT, We1, be1r, We2, Wo1, bo1r, be2r, 1)
    W2stack = jnp.zeros((2 * H, 2), f32)
    W2stack = W2stack.at[:H, 0].set(Wo2[:, 0]).at[H:, 1].set(Wo2[:, 0])

    y1t, r1 = _node1(xp, Wl1, Wr1, bl1r)
    acc1 = _seg_sum(y1t, srcm_s, dstm_s, ztab)
    y2t, r2 = _comb1(acc1, r1, Wl2, Wr2, bl2r, zp0)
    acc2 = _seg_sum(y2t, srcm_s, dstm_s, ztab)
    g = _comb2(acc2, r2, Wo1, zp1)
    ef0 = _edge_feats(g, srcm_e0, dstm_e0)
    ef1 = _edge_feats(g, srcm_e1, dstm_e1)
    o0 = _out_stage(zp0, ef0, W2stack, bo2r)
    o1 = _out_stage(zp1, ef1, W2stack, bo2r)
    return jnp.concatenate(
        [o0[:EQ], o1[:EQ], o0[EQ:], o1[EQ:EQ + (E - EHALF - EQ)]])
